# baseline (speedup 1.0000x reference)
import functools

import numpy as np
import jax
import jax.numpy as jnp
from jax import lax
from jax.experimental import pallas as pl
from jax.experimental.pallas import tpu as pltpu
from jax.experimental.pallas import tpu_sc as plsc

VOCAB = 1000000
EMBED_DIM = 64
PAD_DIM = 128
SEQ = 200
BATCH = 4096
SCALE = 8.0

LANES = 16
NUM_WORKERS = 32
TOTAL_ROWS = BATCH * SEQ
ROWS_PER_WORKER = TOTAL_ROWS // NUM_WORKERS
C = 512
NUM_CHUNKS = ROWS_PER_WORKER // C
NBUF = 3
MAIN_CHUNKS = (NUM_CHUNKS // NBUF) * NBUF
PE_ROWS = 7
PACK_W = 2000


def _positional_encoder(seq_length, embed_dim):
    position = np.arange(seq_length, dtype=np.float32)[:, None]
    div_term = np.exp(
        np.arange(0, embed_dim, 2, dtype=np.float32)[None, :]
        * -(np.log(10000.0) / embed_dim))
    pe = np.zeros((seq_length, embed_dim), dtype=np.float32)
    pe[:, 0::2] = np.sin(position * div_term)
    pe[:, 1::2] = np.cos(position * div_term)
    return pe

_PE = _positional_encoder(SEQ, EMBED_DIM)


PAD_BT = 1600


def _padk_body(t_ref, o_ref):
    o_ref[:, :EMBED_DIM] = t_ref[...]
    o_ref[:, EMBED_DIM:] = jnp.zeros((PAD_BT, PAD_DIM - EMBED_DIM),
                                     jnp.float32)


_padk = pl.pallas_call(
    _padk_body,
    grid=(VOCAB // PAD_BT,),
    in_specs=[pl.BlockSpec((PAD_BT, EMBED_DIM), lambda i: (i, 0))],
    out_specs=pl.BlockSpec((PAD_BT, PAD_DIM), lambda i: (i, 0)),
    out_shape=jax.ShapeDtypeStruct((VOCAB, PAD_DIM), jnp.float32),
)


def _body(x_hbm, table_hbm, pe_hbm, out_hbm,
          idx_all, rows0, rows1, rows2, pe_v,
          gsem0, gsem1, gsem2, osem0, osem1, osem2):
    wid = lax.axis_index("s") * 2 + lax.axis_index("c")
    base = wid * ROWS_PER_WORKER
    p0 = base // BATCH
    rows = (rows0, rows1, rows2)
    gsem = (gsem0, gsem1, gsem2)
    osem = (osem0, osem1, osem2)

    pltpu.sync_copy(x_hbm.at[pl.ds(base, ROWS_PER_WORKER)], idx_all)
    pltpu.sync_copy(pe_hbm.at[pl.ds(p0, PE_ROWS)], pe_v)

    def idx_of(c):
        return idx_all.at[pl.ds(c * C, C)]

    for par in range(NBUF):
        pltpu.async_copy(table_hbm.at[idx_of(par)], rows[par], gsem[par])

    def do_chunk(c, par, prefetch):
        rows_c = rows[par]
        pltpu.make_async_copy(
            table_hbm.at[idx_of(c)], rows_c, gsem[par]).wait()

        g = base + c * C
        prel = g // BATCH - p0
        pe_regs = [pe_v[prel, pl.ds(jj * LANES, LANES)]
                   for jj in range(EMBED_DIM // LANES)]

        def b_body(b, _):
            for jj in range(EMBED_DIM // LANES):
                sl = pl.ds(jj * LANES, LANES)
                rows_c[b, sl] = rows_c[b, sl] * SCALE + pe_regs[jj]
            return ()

        lax.fori_loop(0, C, b_body, (), unroll=8)

        pltpu.async_copy(
            rows_c,
            out_hbm.at[pl.ds(g, C), pl.ds(0, EMBED_DIM)], osem[par])

        if prefetch:
            pprev = (par - 1) % NBUF

            @pl.when((c >= 1) & (c + 2 < NUM_CHUNKS))
            def _():
                pltpu.make_async_copy(
                    rows[pprev],
                    out_hbm.at[pl.ds(0, C), pl.ds(0, EMBED_DIM)],
                    osem[pprev]).wait()
                pltpu.async_copy(
                    table_hbm.at[idx_of(c + 2)], rows[pprev], gsem[pprev])

    def iter_body(i, _):
        for par in range(NBUF):
            do_chunk(i * NBUF + par, par, True)
        return ()

    lax.fori_loop(0, MAIN_CHUNKS // NBUF, iter_body, (), unroll=False)

    for c in range(MAIN_CHUNKS, NUM_CHUNKS):
        do_chunk(c, c % NBUF, False)
    for par in range(NBUF):
        pltpu.make_async_copy(
            rows[par],
            out_hbm.at[pl.ds(0, C), pl.ds(0, EMBED_DIM)],
            osem[par]).wait()


@functools.partial(jax.jit, donate_argnums=())
def kernel(x, table):
    x_flat = jnp.swapaxes(x, 0, 1).reshape(-1) * 2
    t2d = _padk(table).reshape(2 * VOCAB, EMBED_DIM)

    mesh = plsc.VectorSubcoreMesh(core_axis_name="c", subcore_axis_name="s")
    run = pl.kernel(
        _body,
        mesh=mesh,
        out_type=jax.ShapeDtypeStruct((TOTAL_ROWS, PAD_DIM), jnp.float32),
        scratch_types=[
            pltpu.VMEM((ROWS_PER_WORKER,), jnp.int32),
            pltpu.VMEM((C, EMBED_DIM), jnp.float32),
            pltpu.VMEM((C, EMBED_DIM), jnp.float32),
            pltpu.VMEM((C, EMBED_DIM), jnp.float32),
            pltpu.VMEM((PE_ROWS, EMBED_DIM), jnp.float32),
            pltpu.SemaphoreType.DMA,
            pltpu.SemaphoreType.DMA,
            pltpu.SemaphoreType.DMA,
            pltpu.SemaphoreType.DMA,
            pltpu.SemaphoreType.DMA,
            pltpu.SemaphoreType.DMA,
        ],
        compiler_params=pltpu.CompilerParams(use_tc_tiling_on_sc=False),
    )
    out128 = run(x_flat, t2d, jnp.asarray(_PE))
    out_sm = out128[:, :EMBED_DIM]
    return out_sm.reshape(SEQ, BATCH, EMBED_DIM).transpose(1, 0, 2)


# speedup vs baseline: 1.3212x; 1.3212x over previous
import functools

import numpy as np
import jax
import jax.numpy as jnp
from jax import lax
from jax.experimental import pallas as pl
from jax.experimental.pallas import tpu as pltpu
from jax.experimental.pallas import tpu_sc as plsc

VOCAB = 1000000
EMBED_DIM = 64
PAD_DIM = 128
SEQ = 200
BATCH = 4096
SCALE = 8.0

LANES = 16
NUM_WORKERS = 32
TOTAL_ROWS = BATCH * SEQ
ROWS_PER_WORKER = TOTAL_ROWS // NUM_WORKERS
C = 512
NUM_CHUNKS = ROWS_PER_WORKER // C
NBUF = 3
MAIN_CHUNKS = (NUM_CHUNKS // NBUF) * NBUF
PE_ROWS = 7
PACK_W = 2000


def _positional_encoder(seq_length, embed_dim):
    position = np.arange(seq_length, dtype=np.float32)[:, None]
    div_term = np.exp(
        np.arange(0, embed_dim, 2, dtype=np.float32)[None, :]
        * -(np.log(10000.0) / embed_dim))
    pe = np.zeros((seq_length, embed_dim), dtype=np.float32)
    pe[:, 0::2] = np.sin(position * div_term)
    pe[:, 1::2] = np.cos(position * div_term)
    return pe

_PE = _positional_encoder(SEQ, EMBED_DIM)


def _body(x_hbm, table_hbm, pe_hbm, out_hbm,
          idx_all, rows0, rows1, rows2, pe_v,
          gsem0, gsem1, gsem2, osem0, osem1, osem2):
    wid = lax.axis_index("s") * 2 + lax.axis_index("c")
    base = wid * ROWS_PER_WORKER
    p0 = base // BATCH
    rows = (rows0, rows1, rows2)
    gsem = (gsem0, gsem1, gsem2)
    osem = (osem0, osem1, osem2)

    pltpu.sync_copy(x_hbm.at[pl.ds(base, ROWS_PER_WORKER)], idx_all)
    pltpu.sync_copy(pe_hbm.at[pl.ds(p0, PE_ROWS)], pe_v)

    def idx_of(c):
        return idx_all.at[pl.ds(c * C, C)]

    for par in range(NBUF):
        pltpu.async_copy(table_hbm.at[idx_of(par)], rows[par], gsem[par])

    def do_chunk(c, par, prefetch):
        rows_c = rows[par]
        pltpu.make_async_copy(
            table_hbm.at[idx_of(c)], rows_c, gsem[par]).wait()

        g = base + c * C
        prel = g // BATCH - p0
        pe_regs = [pe_v[prel, pl.ds(jj * LANES, LANES)]
                   for jj in range(EMBED_DIM // LANES)]

        def b_body(b, _):
            for jj in range(EMBED_DIM // LANES):
                sl = pl.ds(jj * LANES, LANES)
                rows_c[b, sl] = rows_c[b, sl] * SCALE + pe_regs[jj]
            return ()

        lax.fori_loop(0, C, b_body, (), unroll=8)

        pltpu.async_copy(
            rows_c,
            out_hbm.at[pl.ds(g, C), pl.ds(0, EMBED_DIM)], osem[par])

        if prefetch:
            pprev = (par - 1) % NBUF

            @pl.when((c >= 1) & (c + 2 < NUM_CHUNKS))
            def _():
                pltpu.make_async_copy(
                    rows[pprev],
                    out_hbm.at[pl.ds(0, C), pl.ds(0, EMBED_DIM)],
                    osem[pprev]).wait()
                pltpu.async_copy(
                    table_hbm.at[idx_of(c + 2)], rows[pprev], gsem[pprev])

    def iter_body(i, _):
        for par in range(NBUF):
            do_chunk(i * NBUF + par, par, True)
        return ()

    lax.fori_loop(0, MAIN_CHUNKS // NBUF, iter_body, (), unroll=False)

    for c in range(MAIN_CHUNKS, NUM_CHUNKS):
        do_chunk(c, c % NBUF, False)
    for par in range(NBUF):
        pltpu.make_async_copy(
            rows[par],
            out_hbm.at[pl.ds(0, C), pl.ds(0, EMBED_DIM)],
            osem[par]).wait()


@functools.partial(jax.jit, donate_argnums=())
def kernel(x, table):
    x_flat = jnp.swapaxes(x, 0, 1).reshape(-1)
    t2d = table

    mesh = plsc.VectorSubcoreMesh(core_axis_name="c", subcore_axis_name="s")
    run = pl.kernel(
        _body,
        mesh=mesh,
        out_type=jax.ShapeDtypeStruct((TOTAL_ROWS, PAD_DIM), jnp.float32),
        scratch_types=[
            pltpu.VMEM((ROWS_PER_WORKER,), jnp.int32),
            pltpu.VMEM((C, EMBED_DIM), jnp.float32),
            pltpu.VMEM((C, EMBED_DIM), jnp.float32),
            pltpu.VMEM((C, EMBED_DIM), jnp.float32),
            pltpu.VMEM((PE_ROWS, EMBED_DIM), jnp.float32),
            pltpu.SemaphoreType.DMA,
            pltpu.SemaphoreType.DMA,
            pltpu.SemaphoreType.DMA,
            pltpu.SemaphoreType.DMA,
            pltpu.SemaphoreType.DMA,
            pltpu.SemaphoreType.DMA,
        ],
        compiler_params=pltpu.CompilerParams(use_tc_tiling_on_sc=False),
    )
    out128 = run(x_flat, t2d, jnp.asarray(_PE))
    out_sm = out128[:, :EMBED_DIM]
    return out_sm.reshape(SEQ, BATCH, EMBED_DIM).transpose(1, 0, 2)
